# baseline (speedup 1.0000x reference)
import jax
import jax.numpy as jnp
from jax import lax
from jax.experimental import pallas as pl
from jax.experimental.pallas import tpu as pltpu
from jax.experimental.pallas import tpu_sc as plsc

N_NODES = 10000
D = 128
HALF = D // 2
NUM_GRAPHS = 64
LAYERS = 3

NC = 2
NS = 16

K = 512
ACC_ROWS = 10240
ZCH = 64


def _scatter_body(h2_hbm, srcw_hbm, dstw_hbm, out_hbm,
                  src_v, dst_v, rows_a, rows_b, zbuf, acc, sem_a, sem_b):
    c = lax.axis_index("c")
    s = lax.axis_index("s")
    @pl.loop(0, ZCH)
    def _zrow(i):
        for j in range(HALF // 32):
            zbuf[i, pl.ds(j * 32, 32)] = jnp.zeros((32,), jnp.bfloat16)

    z0 = s * (ACC_ROWS // NS)
    for i in range(ACC_ROWS // NS // ZCH):
        pltpu.sync_copy(zbuf, acc.at[pl.ds(z0 + i * ZCH, ZCH)])

    pltpu.sync_copy(srcw_hbm.at[c, s], src_v)
    pltpu.sync_copy(dstw_hbm.at[s], dst_v)

    plsc.subcore_barrier()

    nch = dstw_hbm.shape[1]

    @pl.loop(0, nch)
    def _chunk(ch):
        pltpu.async_copy(h2_hbm.at[src_v.at[ch]], rows_a, sem_a).wait()
        pltpu.sync_copy(rows_a, acc.at[dst_v.at[ch]], add=True)

    plsc.subcore_barrier()

    for i in range(ACC_ROWS // NS // ZCH):
        off = z0 + i * ZCH
        pltpu.sync_copy(acc.at[pl.ds(off, ZCH)], zbuf)
        pltpu.sync_copy(zbuf, out_hbm.at[c, pl.ds(off, ZCH)])


def _make_scatter(nch):
    mesh = plsc.VectorSubcoreMesh(core_axis_name="c", subcore_axis_name="s",
                                  num_cores=NC, num_subcores=NS)
    return pl.kernel(
        _scatter_body,
        out_type=jax.ShapeDtypeStruct((NC, ACC_ROWS, HALF), jnp.bfloat16),
        mesh=mesh,
        compiler_params=pltpu.CompilerParams(use_tc_tiling_on_sc=False),
        scratch_types=[
            pltpu.VMEM((nch, K), jnp.int32),
            pltpu.VMEM((nch, K), jnp.int32),
            pltpu.VMEM((K, HALF), jnp.bfloat16),
            pltpu.VMEM((K, HALF), jnp.bfloat16),
            pltpu.VMEM((ZCH, HALF), jnp.bfloat16),
            pltpu.VMEM_SHARED((ACC_ROWS, HALF), jnp.bfloat16),
            pltpu.SemaphoreType.DMA,
            pltpu.SemaphoreType.DMA,
        ],
    )


def _tc_layer_body(h_ref, agg_ref, batch_ref, eps_ref,
                   w1_ref, b1_ref, w2_ref, b2_ref, bnw_ref, bnb_ref,
                   pw1_ref, pb1_ref, pw2_ref, pb2_ref,
                   hout_ref, hbf_ref, pooled_ref):
    h = h_ref[...]
    agg = jnp.concatenate(
        [agg_ref[0, :N_NODES, :], agg_ref[1, :N_NODES, :]],
        axis=-1).astype(jnp.float32)
    g = (1.0 + eps_ref[0, 0]) * h + agg
    t = jnp.maximum(jnp.dot(g, w1_ref[...],
                            preferred_element_type=jnp.float32) + b1_ref[...], 0.0)
    g = jnp.dot(t, w2_ref[...], preferred_element_type=jnp.float32) + b2_ref[...]
    mean = jnp.mean(g, axis=0, keepdims=True)
    var = jnp.mean((g - mean) ** 2, axis=0, keepdims=True)
    g = bnw_ref[...] * (g - mean) * lax.rsqrt(var + 1e-5) + bnb_ref[...]
    g = jnp.maximum(g, 0.0)
    hout_ref[...] = g
    gb = g.astype(jnp.bfloat16)
    hbf_ref[pl.ds(0, N_NODES), :] = gb[:, :HALF]
    hbf_ref[pl.ds(N_NODES, N_NODES), :] = gb[:, HALF:]

    r = jnp.maximum(jnp.dot(g, pw1_ref[...],
                            preferred_element_type=jnp.float32) + pb1_ref[...], 0.0)
    oh = (jax.lax.broadcasted_iota(jnp.int32, (NUM_GRAPHS, N_NODES), 0)
          == batch_ref[...]).astype(jnp.float32)
    counts = jnp.sum(oh, axis=1, keepdims=True)
    pr = jnp.dot(oh, r, preferred_element_type=jnp.float32)
    num = jnp.dot(pr, pw2_ref[...],
                  preferred_element_type=jnp.float32) + counts * pb2_ref[...]
    pooled_ref[...] = num / jnp.maximum(counts, 1.0)


_tc_layer = pl.pallas_call(
    _tc_layer_body,
    out_shape=(
        jax.ShapeDtypeStruct((N_NODES, D), jnp.float32),
        jax.ShapeDtypeStruct((2 * N_NODES, HALF), jnp.bfloat16),
        jax.ShapeDtypeStruct((NUM_GRAPHS, 10), jnp.float32),
    ),
)


def kernel(x, edge_index, batch, W1, b1, W2, b2, eps, bnw, bnb, pW1, pb1, pW2, pb2):
    n_edges = edge_index.shape[1]
    per_s = -(-n_edges // NS)
    nch_tot = -(-per_s // K)
    pad = NS * nch_tot * K - n_edges

    src = edge_index[0].astype(jnp.int32)
    dst = edge_index[1].astype(jnp.int32)
    src = jnp.concatenate([src, jnp.zeros((pad,), jnp.int32)])
    dst = jnp.concatenate([dst, jnp.full((pad,), N_NODES, jnp.int32)])
    srcw = src.reshape(NS, nch_tot, K)
    srcw2 = jnp.stack([srcw, srcw + N_NODES])
    dstw = dst.reshape(NS, nch_tot, K)

    scatter = _make_scatter(nch_tot)
    batch2d = batch.astype(jnp.int32).reshape(1, N_NODES)

    h = x
    hb = jnp.concatenate([x[:, :HALF], x[:, HALF:]], axis=0).astype(jnp.bfloat16)
    outs = []
    for l in range(LAYERS):
        aggp = scatter(hb, srcw2, dstw)
        h, hb, pooled = _tc_layer(
            h, aggp, batch2d, eps[l].reshape(1, 1),
            W1[l], b1[l].reshape(1, D), W2[l], b2[l].reshape(1, D),
            bnw[l].reshape(1, D), bnb[l].reshape(1, D),
            pW1[l], pb1[l].reshape(1, D), pW2[l], pb2[l].reshape(1, 10),
        )
        outs.append(pooled)
    return jnp.concatenate(outs, axis=-1)


# speedup vs baseline: 1.2000x; 1.2000x over previous
import jax
import jax.numpy as jnp
from jax import lax
from jax.experimental import pallas as pl
from jax.experimental.pallas import tpu as pltpu
from jax.experimental.pallas import tpu_sc as plsc

N_NODES = 10000
D = 128
HALF = D // 2
NUM_GRAPHS = 64
LAYERS = 3

NC = 2
NS = 16

K = 256
ACC_ROWS = 10240
ZCH = 64


def _scatter_body(h2_hbm, srcw_hbm, dstw_hbm, out_hbm,
                  src_v, dst_v, rows_a, rows_b, zbuf, acc, sem_a, sem_b):
    c = lax.axis_index("c")
    s = lax.axis_index("s")
    @pl.loop(0, ZCH)
    def _zrow(i):
        for j in range(HALF // 32):
            zbuf[i, pl.ds(j * 32, 32)] = jnp.zeros((32,), jnp.bfloat16)

    z0 = s * (ACC_ROWS // NS)
    for i in range(ACC_ROWS // NS // ZCH):
        pltpu.sync_copy(zbuf, acc.at[pl.ds(z0 + i * ZCH, ZCH)])

    pltpu.sync_copy(srcw_hbm.at[c, s], src_v)
    pltpu.sync_copy(dstw_hbm.at[s], dst_v)

    plsc.subcore_barrier()

    nch = dstw_hbm.shape[1]

    @pl.loop(0, nch)
    def _chunk(ch):
        pltpu.async_copy(h2_hbm.at[src_v.at[ch]], rows_a, sem_a).wait()
        pltpu.sync_copy(rows_a, acc.at[dst_v.at[ch]], add=True)

    plsc.subcore_barrier()

    for i in range(ACC_ROWS // NS // ZCH):
        off = z0 + i * ZCH
        pltpu.sync_copy(acc.at[pl.ds(off, ZCH)], zbuf)
        pltpu.sync_copy(zbuf, out_hbm.at[c, pl.ds(off, ZCH)])


def _make_scatter(nch):
    mesh = plsc.VectorSubcoreMesh(core_axis_name="c", subcore_axis_name="s",
                                  num_cores=NC, num_subcores=NS)
    return pl.kernel(
        _scatter_body,
        out_type=jax.ShapeDtypeStruct((NC, ACC_ROWS, HALF), jnp.bfloat16),
        mesh=mesh,
        compiler_params=pltpu.CompilerParams(use_tc_tiling_on_sc=False),
        scratch_types=[
            pltpu.VMEM((nch, K), jnp.int32),
            pltpu.VMEM((nch, K), jnp.int32),
            pltpu.VMEM((K, HALF), jnp.bfloat16),
            pltpu.VMEM((K, HALF), jnp.bfloat16),
            pltpu.VMEM((ZCH, HALF), jnp.bfloat16),
            pltpu.VMEM_SHARED((ACC_ROWS, HALF), jnp.bfloat16),
            pltpu.SemaphoreType.DMA,
            pltpu.SemaphoreType.DMA,
        ],
    )


def _tc_layer_body(h_ref, agg_ref, batch_ref, eps_ref,
                   w1_ref, b1_ref, w2_ref, b2_ref, bnw_ref, bnb_ref,
                   pw1_ref, pb1_ref, pw2_ref, pb2_ref,
                   hout_ref, hbf_ref, pooled_ref):
    h = h_ref[...]
    agg = jnp.concatenate(
        [agg_ref[0, :N_NODES, :], agg_ref[1, :N_NODES, :]],
        axis=-1).astype(jnp.float32)
    g = (1.0 + eps_ref[0, 0]) * h + agg
    t = jnp.maximum(jnp.dot(g, w1_ref[...],
                            preferred_element_type=jnp.float32) + b1_ref[...], 0.0)
    g = jnp.dot(t, w2_ref[...], preferred_element_type=jnp.float32) + b2_ref[...]
    mean = jnp.mean(g, axis=0, keepdims=True)
    var = jnp.mean((g - mean) ** 2, axis=0, keepdims=True)
    g = bnw_ref[...] * (g - mean) * lax.rsqrt(var + 1e-5) + bnb_ref[...]
    g = jnp.maximum(g, 0.0)
    hout_ref[...] = g
    gb = g.astype(jnp.bfloat16)
    hbf_ref[pl.ds(0, N_NODES), :] = gb[:, :HALF]
    hbf_ref[pl.ds(N_NODES, N_NODES), :] = gb[:, HALF:]

    r = jnp.maximum(jnp.dot(g, pw1_ref[...],
                            preferred_element_type=jnp.float32) + pb1_ref[...], 0.0)
    oh = (jax.lax.broadcasted_iota(jnp.int32, (NUM_GRAPHS, N_NODES), 0)
          == batch_ref[...]).astype(jnp.float32)
    counts = jnp.sum(oh, axis=1, keepdims=True)
    pr = jnp.dot(oh, r, preferred_element_type=jnp.float32)
    num = jnp.dot(pr, pw2_ref[...],
                  preferred_element_type=jnp.float32) + counts * pb2_ref[...]
    pooled_ref[...] = num / jnp.maximum(counts, 1.0)


_tc_layer = pl.pallas_call(
    _tc_layer_body,
    out_shape=(
        jax.ShapeDtypeStruct((N_NODES, D), jnp.float32),
        jax.ShapeDtypeStruct((2 * N_NODES, HALF), jnp.bfloat16),
        jax.ShapeDtypeStruct((NUM_GRAPHS, 10), jnp.float32),
    ),
)


def kernel(x, edge_index, batch, W1, b1, W2, b2, eps, bnw, bnb, pW1, pb1, pW2, pb2):
    n_edges = edge_index.shape[1]
    per_s = -(-n_edges // NS)
    nch_tot = -(-per_s // K)
    pad = NS * nch_tot * K - n_edges

    src = edge_index[0].astype(jnp.int32)
    dst = edge_index[1].astype(jnp.int32)
    src = jnp.concatenate([src, jnp.zeros((pad,), jnp.int32)])
    dst = jnp.concatenate([dst, jnp.full((pad,), N_NODES, jnp.int32)])
    srcw = src.reshape(NS, nch_tot, K)
    srcw2 = jnp.stack([srcw, srcw + N_NODES])
    dstw = dst.reshape(NS, nch_tot, K)

    scatter = _make_scatter(nch_tot)
    batch2d = batch.astype(jnp.int32).reshape(1, N_NODES)

    h = x
    hb = jnp.concatenate([x[:, :HALF], x[:, HALF:]], axis=0).astype(jnp.bfloat16)
    outs = []
    for l in range(LAYERS):
        aggp = scatter(hb, srcw2, dstw)
        h, hb, pooled = _tc_layer(
            h, aggp, batch2d, eps[l].reshape(1, 1),
            W1[l], b1[l].reshape(1, D), W2[l], b2[l].reshape(1, D),
            bnw[l].reshape(1, D), bnb[l].reshape(1, D),
            pW1[l], pb1[l].reshape(1, D), pW2[l], pb2[l].reshape(1, 10),
        )
        outs.append(pooled)
    return jnp.concatenate(outs, axis=-1)


# speedup vs baseline: 1.2091x; 1.0075x over previous
import jax
import jax.numpy as jnp
from jax import lax
from jax.experimental import pallas as pl
from jax.experimental.pallas import tpu as pltpu
from jax.experimental.pallas import tpu_sc as plsc

N_NODES = 10000
D = 128
HALF = D // 2
NUM_GRAPHS = 64
LAYERS = 3

NC = 2
NS = 16

K = 256
ACC_ROWS = 10240
ZCH = 64


def _scatter_body(h2_hbm, srcw_hbm, dstw_hbm, out_hbm,
                  src_v, dst_v, rows_a, rows_b, zbuf, acc, sem_a, sem_b):
    c = lax.axis_index("c")
    s = lax.axis_index("s")
    @pl.loop(0, ZCH)
    def _zrow(i):
        for j in range(HALF // 32):
            zbuf[i, pl.ds(j * 32, 32)] = jnp.zeros((32,), jnp.bfloat16)

    z0 = s * (ACC_ROWS // NS)
    for i in range(ACC_ROWS // NS // ZCH):
        pltpu.sync_copy(zbuf, acc.at[pl.ds(z0 + i * ZCH, ZCH)])

    pltpu.sync_copy(srcw_hbm.at[c, s], src_v)
    pltpu.sync_copy(dstw_hbm.at[s], dst_v)

    plsc.subcore_barrier()

    nch = dstw_hbm.shape[1]

    @pl.loop(0, nch)
    def _chunk(ch):
        pltpu.async_copy(h2_hbm.at[src_v.at[ch]], rows_a, sem_a).wait()
        pltpu.sync_copy(rows_a, acc.at[dst_v.at[ch]], add=True)

    plsc.subcore_barrier()

    pltpu.sync_copy(acc.at[pl.ds(z0, ACC_ROWS // NS)],
                    out_hbm.at[c, pl.ds(z0, ACC_ROWS // NS)])


def _make_scatter(nch):
    mesh = plsc.VectorSubcoreMesh(core_axis_name="c", subcore_axis_name="s",
                                  num_cores=NC, num_subcores=NS)
    return pl.kernel(
        _scatter_body,
        out_type=jax.ShapeDtypeStruct((NC, ACC_ROWS, HALF), jnp.bfloat16),
        mesh=mesh,
        compiler_params=pltpu.CompilerParams(use_tc_tiling_on_sc=False),
        scratch_types=[
            pltpu.VMEM((nch, K), jnp.int32),
            pltpu.VMEM((nch, K), jnp.int32),
            pltpu.VMEM((K, HALF), jnp.bfloat16),
            pltpu.VMEM((K, HALF), jnp.bfloat16),
            pltpu.VMEM((ZCH, HALF), jnp.bfloat16),
            pltpu.VMEM_SHARED((ACC_ROWS, HALF), jnp.bfloat16),
            pltpu.SemaphoreType.DMA,
            pltpu.SemaphoreType.DMA,
        ],
    )


def _tc_layer_body(h_ref, agg_ref, batch_ref, eps_ref,
                   w1_ref, b1_ref, w2_ref, b2_ref, bnw_ref, bnb_ref,
                   pw1_ref, pb1_ref, pw2_ref, pb2_ref,
                   hout_ref, hbf_ref, pooled_ref):
    h = h_ref[...]
    agg = jnp.concatenate(
        [agg_ref[0, :N_NODES, :], agg_ref[1, :N_NODES, :]],
        axis=-1).astype(jnp.float32)
    g = (1.0 + eps_ref[0, 0]) * h + agg
    t = jnp.maximum(jnp.dot(g, w1_ref[...],
                            preferred_element_type=jnp.float32) + b1_ref[...], 0.0)
    g = jnp.dot(t, w2_ref[...], preferred_element_type=jnp.float32) + b2_ref[...]
    mean = jnp.mean(g, axis=0, keepdims=True)
    var = jnp.mean((g - mean) ** 2, axis=0, keepdims=True)
    g = bnw_ref[...] * (g - mean) * lax.rsqrt(var + 1e-5) + bnb_ref[...]
    g = jnp.maximum(g, 0.0)
    hout_ref[...] = g
    gb = g.astype(jnp.bfloat16)
    hbf_ref[pl.ds(0, N_NODES), :] = gb[:, :HALF]
    hbf_ref[pl.ds(N_NODES, N_NODES), :] = gb[:, HALF:]

    r = jnp.maximum(jnp.dot(g, pw1_ref[...],
                            preferred_element_type=jnp.float32) + pb1_ref[...], 0.0)
    oh = (jax.lax.broadcasted_iota(jnp.int32, (NUM_GRAPHS, N_NODES), 0)
          == batch_ref[...]).astype(jnp.float32)
    counts = jnp.sum(oh, axis=1, keepdims=True)
    pr = jnp.dot(oh, r, preferred_element_type=jnp.float32)
    num = jnp.dot(pr, pw2_ref[...],
                  preferred_element_type=jnp.float32) + counts * pb2_ref[...]
    pooled_ref[...] = num / jnp.maximum(counts, 1.0)


_tc_layer = pl.pallas_call(
    _tc_layer_body,
    out_shape=(
        jax.ShapeDtypeStruct((N_NODES, D), jnp.float32),
        jax.ShapeDtypeStruct((2 * N_NODES, HALF), jnp.bfloat16),
        jax.ShapeDtypeStruct((NUM_GRAPHS, 10), jnp.float32),
    ),
)


def kernel(x, edge_index, batch, W1, b1, W2, b2, eps, bnw, bnb, pW1, pb1, pW2, pb2):
    n_edges = edge_index.shape[1]
    per_s = -(-n_edges // NS)
    nch_tot = -(-per_s // K)
    pad = NS * nch_tot * K - n_edges

    src = edge_index[0].astype(jnp.int32)
    dst = edge_index[1].astype(jnp.int32)
    src = jnp.concatenate([src, jnp.zeros((pad,), jnp.int32)])
    dst = jnp.concatenate([dst, jnp.full((pad,), N_NODES, jnp.int32)])
    srcw = src.reshape(NS, nch_tot, K)
    srcw2 = jnp.stack([srcw, srcw + N_NODES])
    dstw = dst.reshape(NS, nch_tot, K)

    scatter = _make_scatter(nch_tot)
    batch2d = batch.astype(jnp.int32).reshape(1, N_NODES)

    h = x
    hb = jnp.concatenate([x[:, :HALF], x[:, HALF:]], axis=0).astype(jnp.bfloat16)
    outs = []
    for l in range(LAYERS):
        aggp = scatter(hb, srcw2, dstw)
        h, hb, pooled = _tc_layer(
            h, aggp, batch2d, eps[l].reshape(1, 1),
            W1[l], b1[l].reshape(1, D), W2[l], b2[l].reshape(1, D),
            bnw[l].reshape(1, D), bnb[l].reshape(1, D),
            pW1[l], pb1[l].reshape(1, D), pW2[l], pb2[l].reshape(1, 10),
        )
        outs.append(pooled)
    return jnp.concatenate(outs, axis=-1)


# speedup vs baseline: 1.2097x; 1.0005x over previous
import jax
import jax.numpy as jnp
from jax import lax
from jax.experimental import pallas as pl
from jax.experimental.pallas import tpu as pltpu
from jax.experimental.pallas import tpu_sc as plsc

N_NODES = 10000
D = 128
HALF = D // 2
NUM_GRAPHS = 64
LAYERS = 3

NC = 2
NS = 16

K = 256
ACC_ROWS = 10240
ZCH = 64


def _scatter_body(h2_hbm, srcw_hbm, dstw_hbm, out_hbm,
                  src_v, dst_v, rows_v, zbuf, acc, sem):
    c = lax.axis_index("c")
    s = lax.axis_index("s")
    @pl.loop(0, ZCH)
    def _zrow(i):
        for j in range(HALF // 32):
            zbuf[i, pl.ds(j * 32, 32)] = jnp.zeros((32,), jnp.bfloat16)

    z0 = s * (ACC_ROWS // NS)
    for i in range(ACC_ROWS // NS // ZCH):
        pltpu.sync_copy(zbuf, acc.at[pl.ds(z0 + i * ZCH, ZCH)])

    pltpu.sync_copy(srcw_hbm.at[c, s], src_v)
    pltpu.sync_copy(dstw_hbm.at[s], dst_v)

    plsc.subcore_barrier()

    nch = dstw_hbm.shape[1]

    @pl.loop(0, nch)
    def _chunk(ch):
        pltpu.async_copy(h2_hbm.at[src_v.at[ch]], rows_v, sem).wait()
        pltpu.sync_copy(rows_v, acc.at[dst_v.at[ch]], add=True)

    plsc.subcore_barrier()

    pltpu.sync_copy(acc.at[pl.ds(z0, ACC_ROWS // NS)],
                    out_hbm.at[c, pl.ds(z0, ACC_ROWS // NS)])


def _make_scatter(nch):
    mesh = plsc.VectorSubcoreMesh(core_axis_name="c", subcore_axis_name="s",
                                  num_cores=NC, num_subcores=NS)
    return pl.kernel(
        _scatter_body,
        out_type=jax.ShapeDtypeStruct((NC, ACC_ROWS, HALF), jnp.bfloat16),
        mesh=mesh,
        compiler_params=pltpu.CompilerParams(use_tc_tiling_on_sc=False),
        scratch_types=[
            pltpu.VMEM((nch, K), jnp.int32),
            pltpu.VMEM((nch, K), jnp.int32),
            pltpu.VMEM((K, HALF), jnp.bfloat16),
            pltpu.VMEM((ZCH, HALF), jnp.bfloat16),
            pltpu.VMEM_SHARED((ACC_ROWS, HALF), jnp.bfloat16),
            pltpu.SemaphoreType.DMA,
        ],
    )


def _tc_layer_body(h_ref, agg_ref, batch_ref, eps_ref,
                   w1_ref, b1_ref, w2_ref, b2_ref, bnw_ref, bnb_ref,
                   pw1_ref, pb1_ref, pw2_ref, pb2_ref,
                   hout_ref, hbf_ref, pooled_ref):
    h = h_ref[...]
    agg = jnp.concatenate(
        [agg_ref[0, :N_NODES, :], agg_ref[1, :N_NODES, :]],
        axis=-1).astype(jnp.float32)
    g = (1.0 + eps_ref[0, 0]) * h + agg
    t = jnp.maximum(jnp.dot(g, w1_ref[...],
                            preferred_element_type=jnp.float32) + b1_ref[...], 0.0)
    g = jnp.dot(t, w2_ref[...], preferred_element_type=jnp.float32) + b2_ref[...]
    mean = jnp.mean(g, axis=0, keepdims=True)
    var = jnp.mean((g - mean) ** 2, axis=0, keepdims=True)
    g = bnw_ref[...] * (g - mean) * lax.rsqrt(var + 1e-5) + bnb_ref[...]
    g = jnp.maximum(g, 0.0)
    hout_ref[...] = g
    gb = g.astype(jnp.bfloat16)
    hbf_ref[pl.ds(0, N_NODES), :] = gb[:, :HALF]
    hbf_ref[pl.ds(N_NODES, N_NODES), :] = gb[:, HALF:]

    r = jnp.maximum(jnp.dot(g, pw1_ref[...],
                            preferred_element_type=jnp.float32) + pb1_ref[...], 0.0)
    oh = (jax.lax.broadcasted_iota(jnp.int32, (NUM_GRAPHS, N_NODES), 0)
          == batch_ref[...]).astype(jnp.float32)
    counts = jnp.sum(oh, axis=1, keepdims=True)
    pr = jnp.dot(oh, r, preferred_element_type=jnp.float32)
    num = jnp.dot(pr, pw2_ref[...],
                  preferred_element_type=jnp.float32) + counts * pb2_ref[...]
    pooled_ref[...] = num / jnp.maximum(counts, 1.0)


_tc_layer = pl.pallas_call(
    _tc_layer_body,
    out_shape=(
        jax.ShapeDtypeStruct((N_NODES, D), jnp.float32),
        jax.ShapeDtypeStruct((2 * N_NODES, HALF), jnp.bfloat16),
        jax.ShapeDtypeStruct((NUM_GRAPHS, 10), jnp.float32),
    ),
)


def kernel(x, edge_index, batch, W1, b1, W2, b2, eps, bnw, bnb, pW1, pb1, pW2, pb2):
    n_edges = edge_index.shape[1]
    per_s = -(-n_edges // NS)
    nch_tot = -(-per_s // K)
    pad = NS * nch_tot * K - n_edges

    src = edge_index[0].astype(jnp.int32)
    dst = edge_index[1].astype(jnp.int32)
    src = jnp.concatenate([src, jnp.zeros((pad,), jnp.int32)])
    dst = jnp.concatenate([dst, jnp.full((pad,), N_NODES, jnp.int32)])
    srcw = src.reshape(NS, nch_tot, K)
    srcw2 = jnp.stack([srcw, srcw + N_NODES])
    dstw = dst.reshape(NS, nch_tot, K)

    scatter = _make_scatter(nch_tot)
    batch2d = batch.astype(jnp.int32).reshape(1, N_NODES)

    h = x
    hb = jnp.concatenate([x[:, :HALF], x[:, HALF:]], axis=0).astype(jnp.bfloat16)
    outs = []
    for l in range(LAYERS):
        aggp = scatter(hb, srcw2, dstw)
        h, hb, pooled = _tc_layer(
            h, aggp, batch2d, eps[l].reshape(1, 1),
            W1[l], b1[l].reshape(1, D), W2[l], b2[l].reshape(1, D),
            bnw[l].reshape(1, D), bnb[l].reshape(1, D),
            pW1[l], pb1[l].reshape(1, D), pW2[l], pb2[l].reshape(1, 10),
        )
        outs.append(pooled)
    return jnp.concatenate(outs, axis=-1)
